# baseline (speedup 1.0000x reference)
import jax
import jax.numpy as jnp
import numpy as np
from jax.experimental import pallas as pl
from jax.experimental.pallas import tpu as pltpu
from jax.experimental.pallas import tpu_sc as plsc

B = 4096
V = 100000
D = 64
F = 26
NF = F + 1
NUM_IDX = B * F
GATHER_WINDOW = 128
BS = 512


def _sc_gather(pair_tables, pair_idx):
    mesh = plsc.VectorSubcoreMesh(core_axis_name="core", subcore_axis_name="subcore")

    @pl.kernel(
        out_type=jax.ShapeDtypeStruct((NUM_IDX, 2 * D), pair_tables.dtype),
        mesh=mesh,
    )
    def gather_kernel(x_hbm, i_hbm, o_hbm):
        def body(i_vmem, o_vmem):
            pltpu.sync_copy(x_hbm.at[i_vmem.at[0]], o_vmem)

        pltpu.emit_pipeline(
            body,
            grid=(NUM_IDX // GATHER_WINDOW,),
            in_specs=[pl.BlockSpec((1, GATHER_WINDOW), lambda i: (0, i))],
            out_specs=[pl.BlockSpec((GATHER_WINDOW, 2 * D), lambda i: (i, 0))],
            core_axis_name=("core", "subcore"),
            dimension_semantics=(pltpu.PARALLEL,),
        )(i_hbm, o_hbm)

    return gather_kernel(pair_tables, pair_idx)


def _dense_body(x_ref, emb_ref, par_ref, dw1_, db1_, dw2_, db2_, dw3_, db3_,
                ow1d_, ow1z_, ob1_, ow2_, ob2_, ow3_, ob3_, o_ref):
    f32 = jnp.float32
    x = x_ref[...]
    d = jnp.maximum(jax.lax.dot(x, dw1_[...], preferred_element_type=f32) + db1_[...], 0.0)
    d = jnp.maximum(jax.lax.dot(d, dw2_[...], preferred_element_type=f32) + db2_[...], 0.0)
    d = jnp.maximum(jax.lax.dot(d, dw3_[...], preferred_element_type=f32) + db3_[...], 0.0)
    g = emb_ref[...].reshape(BS, F, 2 * D)
    p = par_ref[...][:, :, None]
    emb = g[:, :, :D] * (1.0 - p) + g[:, :, D:] * p
    a = jnp.concatenate([d[:, None, :], emb], axis=1)
    z = jax.lax.dot_general(
        a, a, (((2,), (2,)), ((0,), (0,))), preferred_element_type=f32
    )
    zf = z.reshape(BS, NF * NF)
    h = (jax.lax.dot(d, ow1d_[...], preferred_element_type=f32)
         + jax.lax.dot(zf, ow1z_[...], preferred_element_type=f32)
         + ob1_[...])
    h = jnp.maximum(h, 0.0)
    h = jnp.maximum(jax.lax.dot(h, ow2_[...], preferred_element_type=f32) + ob2_[...], 0.0)
    o_ref[...] = jax.lax.dot(h, ow3_[...], preferred_element_type=f32) + ob3_[...]


_LI, _LJ = np.triu_indices(NF, k=1)
_PAIR_POS = np.asarray(_LI * NF + _LJ)


def kernel(dense_features, sparse_indices, tables, dw1, db1, dw2, db2, dw3,
           db3, ow1, ob1, ow2, ob2, ow3, ob3):
    pair_tables = tables.reshape(F * V // 2, 2 * D)
    offs = (jnp.arange(F, dtype=jnp.int32) * V)[None, :]
    flat_idx = sparse_indices.astype(jnp.int32) + offs
    pair_idx = jax.lax.shift_right_logical(flat_idx, 1).reshape(1, NUM_IDX)
    parity = jnp.bitwise_and(flat_idx, 1).astype(jnp.float32)
    gathered = _sc_gather(pair_tables, pair_idx)
    emb2 = gathered.reshape(B, F * 2 * D)

    ow1d = ow1[:D]
    ow1z = jnp.zeros((NF * NF, ow1.shape[1]), ow1.dtype).at[_PAIR_POS].set(ow1[D:])

    n_blocks = B // BS
    wspec = lambda shape: pl.BlockSpec(shape, lambda i: (0,) * len(shape))
    out = pl.pallas_call(
        _dense_body,
        grid=(n_blocks,),
        in_specs=[
            pl.BlockSpec((BS, dense_features.shape[1]), lambda i: (i, 0)),
            pl.BlockSpec((BS, F * 2 * D), lambda i: (i, 0)),
            pl.BlockSpec((BS, F), lambda i: (i, 0)),
            wspec(dw1.shape), wspec((1, db1.shape[0])),
            wspec(dw2.shape), wspec((1, db2.shape[0])),
            wspec(dw3.shape), wspec((1, db3.shape[0])),
            wspec(ow1d.shape), wspec(ow1z.shape), wspec((1, ob1.shape[0])),
            wspec(ow2.shape), wspec((1, ob2.shape[0])),
            wspec(ow3.shape), wspec((1, ob3.shape[0])),
        ],
        out_specs=pl.BlockSpec((BS, 1), lambda i: (i, 0)),
        out_shape=jax.ShapeDtypeStruct((B, 1), jnp.float32),
    )(
        dense_features, emb2, parity, dw1, db1[None], dw2, db2[None], dw3,
        db3[None], ow1d, ow1z, ob1[None], ow2, ob2[None], ow3, ob3[None],
    )
    return out
